# baseline (speedup 1.0000x reference)
import functools

import jax
import jax.numpy as jnp
from jax import lax
from jax.experimental import pallas as pl
from jax.experimental.pallas import tpu as pltpu
from jax.experimental.pallas import tpu_sc as plsc

B = 16384
F = 26
D = 32
NC = 2
NS = 16
NW = NC * NS
BLK = 128
NBB = B // BLK
NBLK = F * NBB
BPT = NBLK // NW
L = 16
DT = D // 8
TSZ = 8 * BLK


def _sc_body(cat_hbm, offs_hbm, lanes_hbm, table_hbm, out_hbm,
             idx0, idx1, rows0, rows1, t0, t1, offs_v, lanes_v,
             gsem0, gsem1, wsem0, wsem1):
    cid = lax.axis_index("c")
    sid = lax.axis_index("s")
    wid = sid * NC + cid

    idx_bufs = (idx0, idx1)
    rows_bufs = (rows0, rows1)
    t_bufs = (t0, t1)
    gsems = (gsem0, gsem1)
    wsems = (wsem0, wsem1)

    pltpu.sync_copy(offs_hbm, offs_v)
    pltpu.sync_copy(lanes_hbm, lanes_v)

    base = wid * BPT

    def load_idx(k, buf):
        bid = base + k
        f = bid // NBB
        bblk = bid % NBB
        pltpu.sync_copy(cat_hbm.at[f, pl.ds(bblk * BLK, BLK)], idx_bufs[buf])
        ov = offs_v[f, :]
        for s in range(BLK // L):
            sl = pl.ds(s * L, L)
            idx_bufs[buf][sl] = idx_bufs[buf][sl] + ov

    def fire_gather(buf):
        pltpu.async_copy(table_hbm.at[idx_bufs[buf]], rows_bufs[buf],
                         gsems[buf])

    def finish_block(k, buf, wait_t):
        bid = base + k
        f = bid // NBB
        bblk = bid % NBB
        rows_v = rows_bufs[buf]
        t_v = t_bufs[buf]

        pltpu.make_async_copy(table_hbm.at[idx_bufs[buf]], rows_v,
                              gsems[buf]).wait()

        if wait_t:
            for tr in range(DT):
                pltpu.make_async_copy(
                    t_v.at[tr],
                    out_hbm.at[f, tr, bblk], wsems[buf]).wait()

        for d in range(D):
            i1 = lanes_v[:] * 0 + d
            for lg in range(BLK // L):
                i0 = lanes_v[:] + (lg * L)
                vec = plsc.load_gather(rows_v, [i0, i1])
                t_v[d // 8, d % 8, pl.ds(lg * L, L)] = vec

        for tr in range(DT):
            pltpu.async_copy(t_v.at[tr],
                             out_hbm.at[f, tr, bblk], wsems[buf])

    load_idx(0, 0)
    fire_gather(0)
    load_idx(1, 1)
    fire_gather(1)
    finish_block(0, 0, False)
    load_idx(2, 0)
    fire_gather(0)
    finish_block(1, 1, False)

    def pair_mid(t, carry):
        k = t * 2
        load_idx(k + 1, 1)
        fire_gather(1)
        finish_block(k, 0, True)
        load_idx(k + 2, 0)
        fire_gather(0)
        finish_block(k + 1, 1, True)
        return carry

    lax.fori_loop(1, BPT // 2 - 1, pair_mid, 0)
    k = BPT - 2
    load_idx(k + 1, 1)
    fire_gather(1)
    finish_block(k, 0, True)
    finish_block(k + 1, 1, True)

    for buf in range(2):
        bid = base + BPT - 2 + buf
        f = bid // NBB
        bblk = bid % NBB
        for tr in range(DT):
            pltpu.make_async_copy(
                t_bufs[buf].at[tr],
                out_hbm.at[f, tr, bblk], wsems[buf]).wait()


@functools.partial(
    pl.kernel,
    out_type=jax.ShapeDtypeStruct((F, DT, NBB, 8, BLK), jnp.float32),
    mesh=plsc.VectorSubcoreMesh(core_axis_name="c", subcore_axis_name="s"),
    compiler_params=pltpu.CompilerParams(
        use_tc_tiling_on_sc=False, needs_layout_passes=False),
    scratch_types=[
        pltpu.VMEM((BLK,), jnp.int32),
        pltpu.VMEM((BLK,), jnp.int32),
        pltpu.VMEM((BLK, D), jnp.float32),
        pltpu.VMEM((BLK, D), jnp.float32),
        pltpu.VMEM((DT, 8, BLK), jnp.float32),
        pltpu.VMEM((DT, 8, BLK), jnp.float32),
        pltpu.VMEM((F, L), jnp.int32),
        pltpu.VMEM((L,), jnp.int32),
        pltpu.SemaphoreType.DMA,
        pltpu.SemaphoreType.DMA,
        pltpu.SemaphoreType.DMA,
        pltpu.SemaphoreType.DMA,
    ],
)
def _gather_kernel(cat_hbm, offs_hbm, lanes_hbm, table_hbm, out_hbm,
                   idx0, idx1, rows0, rows1, t0, t1, offs_v, lanes_v,
                   gsem0, gsem1, wsem0, wsem1):
    _sc_body(cat_hbm, offs_hbm, lanes_hbm, table_hbm, out_hbm,
             idx0, idx1, rows0, rows1, t0, t1, offs_v, lanes_v,
             gsem0, gsem1, wsem0, wsem1)


def kernel(categorical_inputs, weights, offsets):
    cat_t = categorical_inputs.astype(jnp.int32).T
    offs16 = jnp.broadcast_to(
        offsets.astype(jnp.int32)[:, None], (F, L))
    lanes = jnp.arange(L, dtype=jnp.int32)
    out6 = _gather_kernel(cat_t, offs16, lanes, weights)
    return out6.transpose(2, 4, 0, 1, 3).reshape(B, F, D)
